# baseline (speedup 1.0000x reference)
import jax
import jax.numpy as jnp
import numpy as np
from jax import lax
from jax.experimental import pallas as pl
from jax.experimental.pallas import tpu as pltpu
from jax.experimental.pallas import tpu_sc as plsc

BATCH = 16384
HIST = 50
EMBED = 32
HIDDEN = 64
VOCAB = 1000000
PADH = 50
RPG = 2
IDXW = RPG * PADH
NC = 2
NS = 16
NW = NC * NS
BPW = BATCH // NW
NCH = BPW // RPG
NBUF = 8

VCHUNK = 512
NFULL = VOCAB // VCHUNK
TAILV = VOCAB - NFULL * VCHUNK
ORPC = VCHUNK * 16 // 128
UROWS = VOCAB * EMBED // 2 // 128
FBUF = 3
FTRIPS = (NFULL + NW - 1) // NW

_DEINT = np.concatenate([np.arange(0, EMBED, 2), np.arange(1, EMBED, 2)])


def _transpose_chunk(in_ref, out_ref, ncols):
    iota = lax.iota(jnp.int32, 16)
    rbase = iota // 8
    cbase = (iota % 8) * 16

    def group(l, carry):
        rvec = rbase + 2 * l
        for p in range(16):
            a = in_ref[2 * p, pl.ds(l * 16, 16)]
            b = in_ref[2 * p + 1, pl.ds(l * 16, 16)]
            u = plsc.bitcast(
                plsc.pack(a, b, format=plsc.PackFormat.INTERLEAVED),
                jnp.int32,
            )
            plsc.store_scatter(out_ref, [rvec, cbase + p], u)
        return carry

    lax.fori_loop(0, ncols // 16, group, 0)


def _sc_format_body(tabT_hbm, tail_hbm, out_hbm, in0, in1, in2, out0, out1,
                    out2, sin, sout):
    ins = (in0, in1, in2)
    outs = (out0, out1, out2)
    cid = lax.axis_index("c")
    sid = lax.axis_index("s")
    wid = sid * NC + cid

    for b in range(FBUF):
        ch = wid + b * NW

        @pl.when(ch < NFULL)
        def _():
            pltpu.async_copy(
                tabT_hbm.at[:, pl.ds(ch * VCHUNK, VCHUNK)], ins[b], sin.at[b]
            )

    def step(i, carry):
        for b in range(FBUF):
            k = i * FBUF + b
            ch = wid + k * NW

            @pl.when(ch < NFULL)
            def _():
                pltpu.make_async_copy(
                    tabT_hbm.at[:, pl.ds(ch * VCHUNK, VCHUNK)],
                    ins[b],
                    sin.at[b],
                ).wait()

                @pl.when(k >= FBUF)
                def _():
                    prev = wid + (k - FBUF) * NW
                    pltpu.make_async_copy(
                        outs[b], out_hbm.at[pl.ds(prev * ORPC, ORPC)], sout.at[b]
                    ).wait()

                _transpose_chunk(ins[b], outs[b], VCHUNK)
                nxt = ch + FBUF * NW

                @pl.when(nxt < NFULL)
                def _():
                    pltpu.async_copy(
                        tabT_hbm.at[:, pl.ds(nxt * VCHUNK, VCHUNK)],
                        ins[b],
                        sin.at[b],
                    )

                pltpu.async_copy(
                    outs[b], out_hbm.at[pl.ds(ch * ORPC, ORPC)], sout.at[b]
                )
        return carry

    lax.fori_loop(0, FTRIPS // FBUF + 1, step, 0)

    nch_w = (NFULL - wid + NW - 1) // NW

    for b in range(FBUF):
        last_k = nch_w - 1 - ((nch_w - 1 - b) % FBUF)

        @pl.when((last_k >= 0) & (last_k % FBUF == b))
        def _():
            ch = wid + last_k * NW
            pltpu.make_async_copy(
                outs[b], out_hbm.at[pl.ds(ch * ORPC, ORPC)], sout.at[b]
            ).wait()

    @pl.when(wid == 0)
    def _():
        pltpu.sync_copy(tail_hbm, ins[0].at[:, pl.ds(0, 128)])
        _transpose_chunk(ins[0], outs[0], TAILV)
        pltpu.sync_copy(
            outs[0].at[pl.ds(0, TAILV // 8)],
            out_hbm.at[pl.ds(NFULL * ORPC, TAILV // 8)],
        )


_sc_format = pl.kernel(
    _sc_format_body,
    out_type=jax.ShapeDtypeStruct((UROWS, 128), jnp.int32),
    mesh=plsc.VectorSubcoreMesh(
        core_axis_name="c", subcore_axis_name="s", num_cores=NC, num_subcores=NS
    ),
    scratch_types=[
        pltpu.VMEM((EMBED, VCHUNK), jnp.float32),
        pltpu.VMEM((EMBED, VCHUNK), jnp.float32),
        pltpu.VMEM((EMBED, VCHUNK), jnp.float32),
        pltpu.VMEM((ORPC, 128), jnp.int32),
        pltpu.VMEM((ORPC, 128), jnp.int32),
        pltpu.VMEM((ORPC, 128), jnp.int32),
        pltpu.SemaphoreType.DMA((FBUF,)),
        pltpu.SemaphoreType.DMA((FBUF,)),
    ],
    compiler_params=pltpu.CompilerParams(
        use_tc_tiling_on_sc=True, needs_layout_passes=False
    ),
)


def _sc_gather_body(table_hbm, idx_hbm, out_hbm, idx_v, rows_v, sums_v, sems):
    cid = lax.axis_index("c")
    sid = lax.axis_index("s")
    wid = sid * NC + cid

    pltpu.sync_copy(idx_hbm.at[wid], idx_v)

    for b in range(NBUF):
        pltpu.async_copy(table_hbm.at[idx_v.at[b]], rows_v.at[b], sems.at[b])

    def step(i, carry):
        for b in range(NBUF):
            ch = i * NBUF + b
            pltpu.make_async_copy(
                table_hbm.at[idx_v.at[ch]], rows_v.at[b], sems.at[b]
            ).wait()
            for r in range(RPG):
                row = ch * RPG + r
                base = r * PADH
                w = plsc.bitcast(rows_v[b, base, 0:16], jnp.bfloat16)
                a0, a1 = plsc.unpack(w, format=plsc.PackFormat.INTERLEAVED)
                for g in range(1, HIST):
                    w = plsc.bitcast(rows_v[b, base + g, 0:16], jnp.bfloat16)
                    lo, hi = plsc.unpack(w, format=plsc.PackFormat.INTERLEAVED)
                    a0 = a0 + lo
                    a1 = a1 + hi
                sums_v[row, 0:16] = a0
                sums_v[row, 16:32] = a1
            nxt = ch + NBUF

            @pl.when(nxt < NCH)
            def _():
                pltpu.async_copy(
                    table_hbm.at[idx_v.at[nxt]], rows_v.at[b], sems.at[b]
                )
        return carry

    lax.fori_loop(0, NCH // NBUF, step, 0)

    pltpu.sync_copy(sums_v, out_hbm.at[pl.ds(wid * BPW, BPW)])


_sc_gather_sum = pl.kernel(
    _sc_gather_body,
    out_type=jax.ShapeDtypeStruct((BATCH, EMBED), jnp.float32),
    mesh=plsc.VectorSubcoreMesh(
        core_axis_name="c", subcore_axis_name="s", num_cores=NC, num_subcores=NS
    ),
    scratch_types=[
        pltpu.VMEM((NCH, IDXW), jnp.int32),
        pltpu.VMEM((NBUF, IDXW, 16), jnp.int32),
        pltpu.VMEM((BPW, EMBED), jnp.float32),
        pltpu.SemaphoreType.DMA((NBUF,)),
    ],
    compiler_params=pltpu.CompilerParams(
        use_tc_tiling_on_sc=False, needs_layout_passes=False
    ),
)


def _mlp_body(s_ref, w1_ref, b1_ref, w2_ref, b2_ref, o_ref):
    m = s_ref[...] * (1.0 / HIST)
    h = jnp.dot(m, w1_ref[...], preferred_element_type=jnp.float32)
    h = jnp.maximum(h + b1_ref[...], 0.0)
    z = jnp.dot(h, w2_ref[...], preferred_element_type=jnp.float32) + b2_ref[...]
    o_ref[...] = 1.0 / (1.0 + jnp.exp(-z))


def kernel(tag_indices, table, W1, b1, W2, b2):
    idx = tag_indices.astype(jnp.int32).reshape(NW, NCH, IDXW)
    tabT = table.T
    tail = jnp.pad(tabT[:, NFULL * VCHUNK :], ((0, 0), (0, 128 - TAILV)))
    packed = _sc_format(tabT, tail)
    tab16 = packed.reshape(VOCAB, 16)
    sums = _sc_gather_sum(tab16, idx)
    w1_perm = W1[_DEINT, :]
    out = pl.pallas_call(
        _mlp_body,
        out_shape=jax.ShapeDtypeStruct((BATCH, 1), jnp.float32),
    )(sums, w1_perm, b1.reshape(1, HIDDEN), W2, b2.reshape(1, 1))
    return out


# speedup vs baseline: 1.0748x; 1.0748x over previous
import jax
import jax.numpy as jnp
import numpy as np
from jax import lax
from jax.experimental import pallas as pl
from jax.experimental.pallas import tpu as pltpu
from jax.experimental.pallas import tpu_sc as plsc

BATCH = 16384
HIST = 50
EMBED = 32
HIDDEN = 64
VOCAB = 1000000
PADH = 50
RPG = 2
IDXW = RPG * PADH
NC = 2
NS = 16
NW = NC * NS
BPW = BATCH // NW
NCH = BPW // RPG
NBUF = 4

VCHUNK = 512
NFULL = VOCAB // VCHUNK
TAILV = VOCAB - NFULL * VCHUNK
ORPC = VCHUNK * 16 // 128
UROWS = VOCAB * EMBED // 2 // 128
FBUF = 3
FTRIPS = (NFULL + NW - 1) // NW

_DEINT = np.concatenate([np.arange(0, EMBED, 2), np.arange(1, EMBED, 2)])


def _transpose_chunk(in_ref, out_ref, ncols):
    iota = lax.iota(jnp.int32, 16)
    rbase = iota // 8
    cbase = (iota % 8) * 16

    def group(l, carry):
        rvec = rbase + 2 * l
        for p in range(16):
            a = in_ref[2 * p, pl.ds(l * 16, 16)]
            b = in_ref[2 * p + 1, pl.ds(l * 16, 16)]
            u = plsc.bitcast(
                plsc.pack(a, b, format=plsc.PackFormat.INTERLEAVED),
                jnp.int32,
            )
            plsc.store_scatter(out_ref, [rvec, cbase + p], u)
        return carry

    lax.fori_loop(0, ncols // 16, group, 0)


def _sc_format_body(tabT_hbm, tail_hbm, out_hbm, in0, in1, in2, out0, out1,
                    out2, sin, sout):
    ins = (in0, in1, in2)
    outs = (out0, out1, out2)
    cid = lax.axis_index("c")
    sid = lax.axis_index("s")
    wid = sid * NC + cid

    for b in range(FBUF):
        ch = wid + b * NW

        @pl.when(ch < NFULL)
        def _():
            pltpu.async_copy(
                tabT_hbm.at[:, pl.ds(ch * VCHUNK, VCHUNK)], ins[b], sin.at[b]
            )

    def step(i, carry):
        for b in range(FBUF):
            k = i * FBUF + b
            ch = wid + k * NW

            @pl.when(ch < NFULL)
            def _():
                pltpu.make_async_copy(
                    tabT_hbm.at[:, pl.ds(ch * VCHUNK, VCHUNK)],
                    ins[b],
                    sin.at[b],
                ).wait()

                @pl.when(k >= FBUF)
                def _():
                    prev = wid + (k - FBUF) * NW
                    pltpu.make_async_copy(
                        outs[b], out_hbm.at[pl.ds(prev * ORPC, ORPC)], sout.at[b]
                    ).wait()

                _transpose_chunk(ins[b], outs[b], VCHUNK)
                nxt = ch + FBUF * NW

                @pl.when(nxt < NFULL)
                def _():
                    pltpu.async_copy(
                        tabT_hbm.at[:, pl.ds(nxt * VCHUNK, VCHUNK)],
                        ins[b],
                        sin.at[b],
                    )

                pltpu.async_copy(
                    outs[b], out_hbm.at[pl.ds(ch * ORPC, ORPC)], sout.at[b]
                )
        return carry

    lax.fori_loop(0, FTRIPS // FBUF + 1, step, 0)

    nch_w = (NFULL - wid + NW - 1) // NW

    for b in range(FBUF):
        last_k = nch_w - 1 - ((nch_w - 1 - b) % FBUF)

        @pl.when((last_k >= 0) & (last_k % FBUF == b))
        def _():
            ch = wid + last_k * NW
            pltpu.make_async_copy(
                outs[b], out_hbm.at[pl.ds(ch * ORPC, ORPC)], sout.at[b]
            ).wait()

    @pl.when(wid == 0)
    def _():
        pltpu.sync_copy(tail_hbm, ins[0].at[:, pl.ds(0, 128)])
        _transpose_chunk(ins[0], outs[0], TAILV)
        pltpu.sync_copy(
            outs[0].at[pl.ds(0, TAILV // 8)],
            out_hbm.at[pl.ds(NFULL * ORPC, TAILV // 8)],
        )


_sc_format = pl.kernel(
    _sc_format_body,
    out_type=jax.ShapeDtypeStruct((UROWS, 128), jnp.int32),
    mesh=plsc.VectorSubcoreMesh(
        core_axis_name="c", subcore_axis_name="s", num_cores=NC, num_subcores=NS
    ),
    scratch_types=[
        pltpu.VMEM((EMBED, VCHUNK), jnp.float32),
        pltpu.VMEM((EMBED, VCHUNK), jnp.float32),
        pltpu.VMEM((EMBED, VCHUNK), jnp.float32),
        pltpu.VMEM((ORPC, 128), jnp.int32),
        pltpu.VMEM((ORPC, 128), jnp.int32),
        pltpu.VMEM((ORPC, 128), jnp.int32),
        pltpu.SemaphoreType.DMA((FBUF,)),
        pltpu.SemaphoreType.DMA((FBUF,)),
    ],
    compiler_params=pltpu.CompilerParams(
        use_tc_tiling_on_sc=True, needs_layout_passes=False
    ),
)


def _sc_gather_body(table_hbm, idx_hbm, out_hbm, idx_v, rows_v, sums_v, sems):
    cid = lax.axis_index("c")
    sid = lax.axis_index("s")
    wid = sid * NC + cid

    pltpu.sync_copy(idx_hbm.at[wid], idx_v)

    for b in range(NBUF):
        pltpu.async_copy(table_hbm.at[idx_v.at[b]], rows_v.at[b], sems.at[b])

    def step(i, carry):
        for b in range(NBUF):
            ch = i * NBUF + b
            pltpu.make_async_copy(
                table_hbm.at[idx_v.at[ch]], rows_v.at[b], sems.at[b]
            ).wait()
            for r in range(RPG):
                row = ch * RPG + r
                base = r * PADH
                w = plsc.bitcast(rows_v[b, base, 0:16], jnp.bfloat16)
                a0, a1 = plsc.unpack(w, format=plsc.PackFormat.INTERLEAVED)
                for g in range(1, HIST):
                    w = plsc.bitcast(rows_v[b, base + g, 0:16], jnp.bfloat16)
                    lo, hi = plsc.unpack(w, format=plsc.PackFormat.INTERLEAVED)
                    a0 = a0 + lo
                    a1 = a1 + hi
                sums_v[row, 0:16] = a0
                sums_v[row, 16:32] = a1
            nxt = ch + NBUF

            @pl.when(nxt < NCH)
            def _():
                pltpu.async_copy(
                    table_hbm.at[idx_v.at[nxt]], rows_v.at[b], sems.at[b]
                )
        return carry

    lax.fori_loop(0, NCH // NBUF, step, 0)

    pltpu.sync_copy(sums_v, out_hbm.at[pl.ds(wid * BPW, BPW)])


_sc_gather_sum = pl.kernel(
    _sc_gather_body,
    out_type=jax.ShapeDtypeStruct((BATCH, EMBED), jnp.float32),
    mesh=plsc.VectorSubcoreMesh(
        core_axis_name="c", subcore_axis_name="s", num_cores=NC, num_subcores=NS
    ),
    scratch_types=[
        pltpu.VMEM((NCH, IDXW), jnp.int32),
        pltpu.VMEM((NBUF, IDXW, 16), jnp.int32),
        pltpu.VMEM((BPW, EMBED), jnp.float32),
        pltpu.SemaphoreType.DMA((NBUF,)),
    ],
    compiler_params=pltpu.CompilerParams(
        use_tc_tiling_on_sc=False, needs_layout_passes=False
    ),
)


def _mlp_body(s_ref, w1_ref, b1_ref, w2_ref, b2_ref, o_ref):
    m = s_ref[...] * (1.0 / HIST)
    h = jnp.dot(m, w1_ref[...], preferred_element_type=jnp.float32)
    h = jnp.maximum(h + b1_ref[...], 0.0)
    z = jnp.dot(h, w2_ref[...], preferred_element_type=jnp.float32) + b2_ref[...]
    o_ref[...] = 1.0 / (1.0 + jnp.exp(-z))


def kernel(tag_indices, table, W1, b1, W2, b2):
    idx = tag_indices.astype(jnp.int32).reshape(NW, NCH, IDXW)
    tabT = table.T
    tail = jnp.pad(tabT[:, NFULL * VCHUNK :], ((0, 0), (0, 128 - TAILV)))
    packed = _sc_format(tabT, tail)
    tab16 = packed.reshape(VOCAB, 16)
    sums = _sc_gather_sum(tab16, idx)
    w1_perm = W1[_DEINT, :]
    out = pl.pallas_call(
        _mlp_body,
        out_shape=jax.ShapeDtypeStruct((BATCH, 1), jnp.float32),
    )(sums, w1_perm, b1.reshape(1, HIDDEN), W2, b2.reshape(1, 1))
    return out
